# Initial kernel scaffold; baseline (speedup 1.0000x reference)
import jax
import jax.numpy as jnp
from jax.experimental import pallas as pl


def kernel(x, edge_index, edge_attr, batch, params):
    raise NotImplementedError("write your pallas kernel here")


# speedup vs baseline: 14.9697x; 14.9697x over previous
import functools
import math

import jax
import jax.numpy as jnp
from jax import lax
from jax.experimental import pallas as pl
from jax.experimental.pallas import tpu as pltpu
from jax.experimental.pallas import tpu_sc as plsc

N = 10000
E = 320000
DN = 128
DE = 16
G = 64

NTILES = 32
PER_TILE = E // NTILES
GCH = 200
SCH = 200
ROWS_PER_TILE = N // 16

_P = lax.Precision.HIGHEST


def _dot(a, b):
    return jnp.dot(a, b, precision=_P, preferred_element_type=jnp.float32)


def _sc_gather2(tbl_a, idx_a, tbl_b, idx_b):
    Da = tbl_a.shape[1]
    Db = tbl_b.shape[1]
    nch = PER_TILE // GCH
    mesh = plsc.VectorSubcoreMesh(core_axis_name="c", subcore_axis_name="s")

    @functools.partial(
        pl.kernel,
        mesh=mesh,
        out_type=[
            jax.ShapeDtypeStruct((E, Da), jnp.float32),
            jax.ShapeDtypeStruct((E, Db), jnp.float32),
        ],
        scratch_types=[
            pltpu.VMEM((GCH,), jnp.int32),
            pltpu.VMEM((GCH,), jnp.int32),
            pltpu.VMEM((GCH, Da), jnp.float32),
            pltpu.VMEM((GCH, Db), jnp.float32),
            pltpu.SemaphoreType.DMA,
            pltpu.SemaphoreType.DMA,
        ],
    )
    def k(ta, ia, tb, ib, oa, ob, iva, ivb, ra, rb, sa, sb):
        wid = lax.axis_index("s") * 2 + lax.axis_index("c")
        base = wid * PER_TILE

        @pl.loop(0, nch)
        def _(j):
            off = base + j * GCH
            pltpu.sync_copy(ia.at[pl.ds(off, GCH)], iva)
            pltpu.sync_copy(ib.at[pl.ds(off, GCH)], ivb)
            ca = pltpu.async_copy(ta.at[iva], ra, sa)
            cb = pltpu.async_copy(tb.at[ivb], rb, sb)
            ca.wait()
            cb.wait()
            pltpu.sync_copy(ra, oa.at[pl.ds(off, GCH)])
            pltpu.sync_copy(rb, ob.at[pl.ds(off, GCH)])

    return k(tbl_a, idx_a, tbl_b, idx_b)


_DT = 10
_DR = N // _DT


def _sc_scatter1(vals, dst, zeros):
    D = vals.shape[1]
    nch = PER_TILE // SCH
    mesh = plsc.VectorSubcoreMesh(core_axis_name="c", subcore_axis_name="s")

    @functools.partial(
        pl.kernel,
        mesh=mesh,
        out_type=jax.ShapeDtypeStruct((2 * N, D), jnp.float32),
        scratch_types=[
            pltpu.VMEM((SCH,), jnp.int32),
            pltpu.VMEM((SCH, D), jnp.float32),
            pltpu.VMEM_SHARED((N, D), jnp.float32),
        ],
    )
    def k(v_h, dst_h, z_h, p_h, ibuf, vbuf, acc):
        c = lax.axis_index("c")
        s = lax.axis_index("s")
        wid = s * 2 + c

        @pl.when(s < _DT)
        def _():
            rbase = s * _DR
            pltpu.sync_copy(z_h.at[pl.ds(rbase, _DR)], acc.at[pl.ds(rbase, _DR)])

        plsc.subcore_barrier()

        base = wid * PER_TILE

        @pl.loop(0, nch)
        def _(j):
            off = base + j * SCH
            pltpu.sync_copy(dst_h.at[pl.ds(off, SCH)], ibuf)
            pltpu.sync_copy(v_h.at[pl.ds(off, SCH)], vbuf)
            pltpu.sync_copy(vbuf, acc.at[ibuf], add=True)

        plsc.subcore_barrier()

        @pl.when(s < _DT)
        def _():
            rbase = s * _DR
            pltpu.sync_copy(acc.at[pl.ds(rbase, _DR)],
                            p_h.at[pl.ds(c * N + rbase, _DR)])

    return k(vals, dst, zeros)


def _sc_scatter(wv, exo, dst):
    pwv = _sc_scatter1(wv, dst, jnp.zeros((N, DN), jnp.float32))
    pex = _sc_scatter1(exo, dst, jnp.zeros((N, 16), jnp.float32))
    return pwv, pex


_NB = 2000
_NBLK = N // _NB
_EB = 4000


def _node_linear(inp, Wcat, bcat):
    def body(x_ref, w_ref, b_ref, q_ref, kv_ref, s_ref):
        y = _dot(x_ref[...], w_ref[...]) + b_ref[...]
        q_ref[...] = y[:, :DN]
        kv_ref[...] = y[:, DN:3 * DN]
        s_ref[...] = y[:, 3 * DN:]

    return pl.pallas_call(
        body,
        grid=(_NBLK,),
        in_specs=[
            pl.BlockSpec((_NB, DN), lambda i: (i, 0)),
            pl.BlockSpec((DN, 4 * DN), lambda i: (0, 0)),
            pl.BlockSpec((1, 4 * DN), lambda i: (0, 0)),
        ],
        out_specs=[
            pl.BlockSpec((_NB, DN), lambda i: (i, 0)),
            pl.BlockSpec((_NB, 2 * DN), lambda i: (i, 0)),
            pl.BlockSpec((_NB, DN), lambda i: (i, 0)),
        ],
        out_shape=[
            jax.ShapeDtypeStruct((N, DN), jnp.float32),
            jax.ShapeDtypeStruct((N, 2 * DN), jnp.float32),
            jax.ShapeDtypeStruct((N, DN), jnp.float32),
        ],
    )(inp, Wcat, bcat)


def _head_expand():
    r = lax.broadcasted_iota(jnp.int32, (8, DN), 0)
    cdiv = lax.broadcasted_iota(jnp.int32, (8, DN), 1) // 16
    return (r == cdiv).astype(jnp.float32)


def _edge_math1(qd, kvs, ea, We1):
    def body(q_ref, kv_ref, a_ref, w_ref, wv_ref, ex_ref):
        e = _dot(a_ref[...], w_ref[...])
        kv = kv_ref[...]
        kj = kv[:, :DN] + e
        vj = kv[:, DN:] + e
        prod = q_ref[...] * kj
        r16 = lax.broadcasted_iota(jnp.int32, (DN, 8), 0) // 16
        hcol = lax.broadcasted_iota(jnp.int32, (DN, 8), 1)
        hm = jnp.where(r16 == hcol, 0.25, 0.0)
        ex = jnp.exp(_dot(prod, hm))
        wv_ref[...] = vj * _dot(ex, _head_expand())
        pad = lax.broadcasted_iota(jnp.int32, (8, 16), 0)
        padc = lax.broadcasted_iota(jnp.int32, (8, 16), 1)
        ex_ref[...] = _dot(ex, (pad == padc).astype(jnp.float32))

    return pl.pallas_call(
        body,
        grid=(E // _EB,),
        in_specs=[
            pl.BlockSpec((_EB, DN), lambda i: (i, 0)),
            pl.BlockSpec((_EB, 2 * DN), lambda i: (i, 0)),
            pl.BlockSpec((_EB, DE), lambda i: (i, 0)),
            pl.BlockSpec((DE, DN), lambda i: (0, 0)),
        ],
        out_specs=[
            pl.BlockSpec((_EB, DN), lambda i: (i, 0)),
            pl.BlockSpec((_EB, 16), lambda i: (i, 0)),
        ],
        out_shape=[
            jax.ShapeDtypeStruct((E, DN), jnp.float32),
            jax.ShapeDtypeStruct((E, 16), jnp.float32),
        ],
    )(qd, kvs, ea, We1)


def _edge_math2(qd, kvs, ea, We2):
    inv = 1.0 / math.sqrt(128.0)

    def body(q_ref, kv_ref, a_ref, w_ref, wv_ref, ex_ref):
        e = _dot(a_ref[...], w_ref[...])
        kv = kv_ref[...]
        kj = kv[:, :DN] + e
        vj = kv[:, DN:] + e
        prod = q_ref[...] * kj
        ones = jnp.full((DN, 1), inv, jnp.float32)
        ex = jnp.exp(_dot(prod, ones))
        wv_ref[...] = vj * ex
        lane = lax.broadcasted_iota(jnp.int32, (_EB, 16), 1)
        ex_ref[...] = jnp.where(lane == 0, ex, 0.0)

    return pl.pallas_call(
        body,
        grid=(E // _EB,),
        in_specs=[
            pl.BlockSpec((_EB, DN), lambda i: (i, 0)),
            pl.BlockSpec((_EB, 2 * DN), lambda i: (i, 0)),
            pl.BlockSpec((_EB, DE), lambda i: (i, 0)),
            pl.BlockSpec((DE, DN), lambda i: (0, 0)),
        ],
        out_specs=[
            pl.BlockSpec((_EB, DN), lambda i: (i, 0)),
            pl.BlockSpec((_EB, 16), lambda i: (i, 0)),
        ],
        out_shape=[
            jax.ShapeDtypeStruct((E, DN), jnp.float32),
            jax.ShapeDtypeStruct((E, 16), jnp.float32),
        ],
    )(qd, kvs, ea, We2)


def _norm1_proj2(pwv, pex, s1, Wcat2, bcat2):
    def body(w0, w1, e0, e1, s_ref, wc, bc, q_ref, kv_ref, so_ref):
        den8 = (e0[...] + e1[...])[:, :8]
        dexp = _dot(den8, _head_expand()) + 1e-16
        h = (w0[...] + w1[...]) / dexp + s_ref[...]
        h = jnp.where(h >= 0, h, 0.01 * h)
        y = _dot(h, wc[...]) + bc[...]
        q_ref[...] = y[:, :DN]
        kv_ref[...] = y[:, DN:3 * DN]
        so_ref[...] = y[:, 3 * DN:]

    return pl.pallas_call(
        body,
        grid=(_NBLK,),
        in_specs=[
            pl.BlockSpec((_NB, DN), lambda i: (i, 0)),
            pl.BlockSpec((_NB, DN), lambda i: (i + _NBLK, 0)),
            pl.BlockSpec((_NB, 16), lambda i: (i, 0)),
            pl.BlockSpec((_NB, 16), lambda i: (i + _NBLK, 0)),
            pl.BlockSpec((_NB, DN), lambda i: (i, 0)),
            pl.BlockSpec((DN, 4 * DN), lambda i: (0, 0)),
            pl.BlockSpec((1, 4 * DN), lambda i: (0, 0)),
        ],
        out_specs=[
            pl.BlockSpec((_NB, DN), lambda i: (i, 0)),
            pl.BlockSpec((_NB, 2 * DN), lambda i: (i, 0)),
            pl.BlockSpec((_NB, DN), lambda i: (i, 0)),
        ],
        out_shape=[
            jax.ShapeDtypeStruct((N, DN), jnp.float32),
            jax.ShapeDtypeStruct((N, 2 * DN), jnp.float32),
            jax.ShapeDtypeStruct((N, DN), jnp.float32),
        ],
    )(pwv, pwv, pex, pex, s1, Wcat2, bcat2)


def _norm2_pool(pwv, pex, s2, batch2d):
    def body(w0, w1, e0, e1, s_ref, b_ref, h_ref, sum_ref, cnt_ref):
        den = (e0[...] + e1[...])[:, 0:1] + 1e-16
        h2 = (w0[...] + w1[...]) / den + s_ref[...]
        h_ref[...] = h2
        gidx = lax.broadcasted_iota(jnp.int32, (G, _NB), 0)
        onehot = (jnp.broadcast_to(b_ref[0], (G, _NB)) == gidx).astype(jnp.float32)
        psum = _dot(onehot, h2)
        pcnt = jnp.broadcast_to(jnp.sum(onehot, axis=1, keepdims=True), (G, DN))

        @pl.when(pl.program_id(0) == 0)
        def _():
            sum_ref[...] = jnp.zeros_like(sum_ref)
            cnt_ref[...] = jnp.zeros_like(cnt_ref)

        sum_ref[...] += psum
        cnt_ref[...] += pcnt

    return pl.pallas_call(
        body,
        grid=(_NBLK,),
        in_specs=[
            pl.BlockSpec((_NB, DN), lambda i: (i, 0)),
            pl.BlockSpec((_NB, DN), lambda i: (i + _NBLK, 0)),
            pl.BlockSpec((_NB, 16), lambda i: (i, 0)),
            pl.BlockSpec((_NB, 16), lambda i: (i + _NBLK, 0)),
            pl.BlockSpec((_NB, DN), lambda i: (i, 0)),
            pl.BlockSpec((1, 1, _NB), lambda i: (i, 0, 0)),
        ],
        out_specs=[
            pl.BlockSpec((_NB, DN), lambda i: (i, 0)),
            pl.BlockSpec((G, DN), lambda i: (0, 0)),
            pl.BlockSpec((G, DN), lambda i: (0, 0)),
        ],
        out_shape=[
            jax.ShapeDtypeStruct((N, DN), jnp.float32),
            jax.ShapeDtypeStruct((G, DN), jnp.float32),
            jax.ShapeDtypeStruct((G, DN), jnp.float32),
        ],
    )(pwv, pwv, pex, pex, s2, batch2d)


def _mlp_prep(h2, sums, cnt, batch2d, Wa, Wb, Wd):
    def body(h_ref, sum_ref, cnt_ref, b_ref, wa, wb, wd, pa_ref, pb_ref):
        pooled = sum_ref[...] / jnp.maximum(cnt_ref[...], 1.0)
        pg = _dot(pooled, wd[...])
        gidx = lax.broadcasted_iota(jnp.int32, (G, _NB), 0)
        onehot = (jnp.broadcast_to(b_ref[0], (G, _NB)) == gidx).astype(jnp.float32)
        nodepg = lax.dot_general(onehot, pg, (((0,), (0,)), ((), ())),
                                 precision=_P, preferred_element_type=jnp.float32)
        h = h_ref[...]
        pa_ref[...] = _dot(h, wa[...]) + nodepg
        pb_ref[...] = _dot(h, wb[...])

    return pl.pallas_call(
        body,
        grid=(_NBLK,),
        in_specs=[
            pl.BlockSpec((_NB, DN), lambda i: (i, 0)),
            pl.BlockSpec((G, DN), lambda i: (0, 0)),
            pl.BlockSpec((G, DN), lambda i: (0, 0)),
            pl.BlockSpec((1, 1, _NB), lambda i: (i, 0, 0)),
            pl.BlockSpec((DN, DN), lambda i: (0, 0)),
            pl.BlockSpec((DN, DN), lambda i: (0, 0)),
            pl.BlockSpec((DN, DN), lambda i: (0, 0)),
        ],
        out_specs=[
            pl.BlockSpec((_NB, DN), lambda i: (i, 0)),
            pl.BlockSpec((_NB, DN), lambda i: (i, 0)),
        ],
        out_shape=[
            jax.ShapeDtypeStruct((N, DN), jnp.float32),
            jax.ShapeDtypeStruct((N, DN), jnp.float32),
        ],
    )(h2, sums, cnt, batch2d, Wa, Wb, Wd)


def _mlp_final(paS, pbD, ea, Wc, bm1, Wm2, bm2):
    def body(a_ref, b_ref, e_ref, wc, b1, w2, b2, o_ref):
        hid = a_ref[...] + b_ref[...] + _dot(e_ref[...], wc[...]) + b1[...]
        hid = jnp.maximum(hid, 0.0)
        o_ref[...] = _dot(hid, w2[...]) + b2[...]

    return pl.pallas_call(
        body,
        grid=(E // _EB,),
        in_specs=[
            pl.BlockSpec((_EB, DN), lambda i: (i, 0)),
            pl.BlockSpec((_EB, DN), lambda i: (i, 0)),
            pl.BlockSpec((_EB, DE), lambda i: (i, 0)),
            pl.BlockSpec((DE, DN), lambda i: (0, 0)),
            pl.BlockSpec((1, DN), lambda i: (0, 0)),
            pl.BlockSpec((DN, 1), lambda i: (0, 0)),
            pl.BlockSpec((1, 1), lambda i: (0, 0)),
        ],
        out_specs=pl.BlockSpec((_EB, 1), lambda i: (i, 0)),
        out_shape=jax.ShapeDtypeStruct((E, 1), jnp.float32),
    )(paS, pbD, ea, Wc, bm1, Wm2, bm2)


def kernel(x, edge_index, edge_attr, batch, params):
    p = params
    src = edge_index[0]
    dst = edge_index[1]

    Wcat1 = jnp.concatenate([p['Wq1'], p['Wk1'], p['Wv1'], p['Ws1']], axis=1)
    bcat1 = jnp.concatenate([p['bq1'], p['bk1'], p['bv1'], p['bs1']])[None, :]
    q1, kv1, s1 = _node_linear(x, Wcat1, bcat1)

    qd1, kvs1 = _sc_gather2(q1, dst, kv1, src)
    wv1, exo1 = _edge_math1(qd1, kvs1, edge_attr, p['We1'])
    pwv1, pex1 = _sc_scatter(wv1, exo1, dst)

    Wcat2 = jnp.concatenate([p['Wq2'], p['Wk2'], p['Wv2'], p['Ws2']], axis=1)
    bcat2 = jnp.concatenate([p['bq2'], p['bk2'], p['bv2'], p['bs2']])[None, :]
    q2, kv2, s2 = _norm1_proj2(pwv1, pex1, s1, Wcat2, bcat2)

    qd2, kvs2 = _sc_gather2(q2, dst, kv2, src)
    wv2, exo2 = _edge_math2(qd2, kvs2, edge_attr, p['We2'])
    pwv2, pex2 = _sc_scatter(wv2, exo2, dst)

    batch3d = batch.reshape(_NBLK, 1, _NB)
    h2, sums, cnt = _norm2_pool(pwv2, pex2, s2, batch3d)

    Wm1 = p['Wm1']
    Wa = Wm1[:DN]
    Wb = Wm1[DN:2 * DN]
    Wc = Wm1[2 * DN:2 * DN + DE]
    Wd = Wm1[2 * DN + DE:]
    pa, pb = _mlp_prep(h2, sums, cnt, batch3d, Wa, Wb, Wd)

    paS, pbD = _sc_gather2(pa, src, pb, dst)
    return _mlp_final(paS, pbD, edge_attr, Wc, p['bm1'][None, :],
                      p['Wm2'], p['bm2'][None, :])
